# baseline (speedup 1.0000x reference)
import jax
import jax.numpy as jnp
from jax import lax
from jax.experimental import pallas as pl
from jax.experimental.pallas import tpu as pltpu
from jax.experimental.pallas import tpu_sc as plsc

BATCH = 16384
EMB_D = 32
PACK = 4
PACK_D = PACK * EMB_D

NUM_CORES = 2
NUM_SUBCORES = 16
NW = NUM_CORES * NUM_SUBCORES
B_PER_W = BATCH // NW
CHUNK = 128
NCHUNK = B_PER_W // CHUNK


def _sc_gather_body(users_hbm, items_hbm, gu_t, gi_t, mu_t, mi_t,
                    gu_o, gi_o, mu_o, mi_o,
                    idx_u, idx_i, bu, bi, bmu, bmi, sem):
    wid = lax.axis_index("s") * NUM_CORES + lax.axis_index("c")
    base = wid * B_PER_W
    row0 = wid * NCHUNK
    pltpu.sync_copy(users_hbm.at[pl.ds(row0, NCHUNK)], idx_u)
    pltpu.sync_copy(items_hbm.at[pl.ds(row0, NCHUNK)], idx_i)
    for c in range(NCHUNK):
        cps = [
            pltpu.async_copy(gu_t.at[idx_u.at[c]], bu, sem),
            pltpu.async_copy(gi_t.at[idx_i.at[c]], bi, sem),
            pltpu.async_copy(mu_t.at[idx_u.at[c]], bmu, sem),
            pltpu.async_copy(mi_t.at[idx_i.at[c]], bmi, sem),
        ]
        for cp in cps:
            cp.wait()
        dst = pl.ds(base + c * CHUNK, CHUNK)
        pltpu.sync_copy(bu, gu_o.at[dst])
        pltpu.sync_copy(bi, gi_o.at[dst])
        pltpu.sync_copy(bmu, mu_o.at[dst])
        pltpu.sync_copy(bmi, mi_o.at[dst])


@jax.jit
def _sc_gather(users, items, gu_t, gi_t, mu_t, mi_t):
    packed = jax.ShapeDtypeStruct((BATCH, PACK_D), jnp.float32)
    mesh = plsc.VectorSubcoreMesh(core_axis_name="c", subcore_axis_name="s")
    f = pl.kernel(
        _sc_gather_body,
        out_type=(packed, packed, packed, packed),
        mesh=mesh,
        scratch_types=[
            pltpu.VMEM((NCHUNK, CHUNK), jnp.int32),
            pltpu.VMEM((NCHUNK, CHUNK), jnp.int32),
            pltpu.VMEM((CHUNK, PACK_D), jnp.float32),
            pltpu.VMEM((CHUNK, PACK_D), jnp.float32),
            pltpu.VMEM((CHUNK, PACK_D), jnp.float32),
            pltpu.VMEM((CHUNK, PACK_D), jnp.float32),
            pltpu.SemaphoreType.DMA,
        ],
    )
    return f(users, items, gu_t, gi_t, mu_t, mi_t)


def _extract(packed, q):
    out = jnp.zeros(packed.shape[:1] + (EMB_D,), jnp.float32)
    for k in range(PACK):
        out = jnp.where(q == k, packed[:, k * EMB_D:(k + 1) * EMB_D], out)
    return out


def _tc_dense_body(pu, pi, pmu, pmi, qu, qi, W1, b1, W2, b2, Wf, bf, out):
    qu_v = qu[...]
    qi_v = qi[...]
    gu = _extract(pu[...], qu_v)
    gi = _extract(pi[...], qi_v)
    mu = _extract(pmu[...], qu_v)
    mi = _extract(pmi[...], qi_v)
    prod = gu * gi
    x1 = (jnp.dot(mu, W1[0:32, :], preferred_element_type=jnp.float32)
          + jnp.dot(mi, W1[32:64, :], preferred_element_type=jnp.float32)
          + b1[...])
    h = jnp.maximum(x1, 0.0)
    m2 = jnp.dot(h, W2[...], preferred_element_type=jnp.float32) + b2[...]
    logit = (jnp.dot(prod, Wf[0:32, :], preferred_element_type=jnp.float32)
             + jnp.dot(m2, Wf[32:96, :], preferred_element_type=jnp.float32)
             + bf[...])
    out[...] = jax.nn.sigmoid(logit)


@jax.jit
def _tc_dense(pu, pi, pmu, pmi, qu, qi, W1, b1, W2, b2, Wf, bf):
    BLK = 4096
    grid = (BATCH // BLK,)
    pk_spec = pl.BlockSpec((BLK, PACK_D), lambda i: (i, 0))
    q_spec = pl.BlockSpec((BLK, 1), lambda i: (i, 0))
    full = lambda shape: pl.BlockSpec(shape, lambda i: tuple(0 for _ in shape))
    return pl.pallas_call(
        _tc_dense_body,
        grid=grid,
        in_specs=[
            pk_spec, pk_spec, pk_spec, pk_spec, q_spec, q_spec,
            full((64, 128)), full((128,)), full((128, 64)), full((64,)),
            full((96, 1)), full((1,)),
        ],
        out_specs=pl.BlockSpec((BLK, 1), lambda i: (i, 0)),
        out_shape=jax.ShapeDtypeStruct((BATCH, 1), jnp.float32),
    )(pu, pi, pmu, pmi, qu, qi, W1, b1, W2, b2, Wf, bf)


def kernel(X, gmf_user_emb, gmf_item_emb, mlp_user_emb, mlp_item_emb,
           W1, b1, W2, b2, Wf, bf):
    users = X[:, 0].astype(jnp.int32)
    items = X[:, 1].astype(jnp.int32)
    pu_idx = (users // PACK).reshape(NW * NCHUNK, CHUNK)
    pi_idx = (items // PACK).reshape(NW * NCHUNK, CHUNK)
    qu = (users % PACK).reshape(BATCH, 1)
    qi = (items % PACK).reshape(BATCH, 1)
    NPACKED = gmf_user_emb.shape[0] // PACK
    gu_t = gmf_user_emb.reshape(NPACKED, PACK_D)
    gi_t = gmf_item_emb.reshape(NPACKED, PACK_D)
    mu_t = mlp_user_emb.reshape(NPACKED, PACK_D)
    mi_t = mlp_item_emb.reshape(NPACKED, PACK_D)
    pu, pi, pmu, pmi = _sc_gather(pu_idx, pi_idx, gu_t, gi_t, mu_t, mi_t)
    return _tc_dense(pu, pi, pmu, pmi, qu, qi, W1, b1, W2, b2, Wf, bf)


# speedup vs baseline: 3.8773x; 3.8773x over previous
import jax
import jax.numpy as jnp
from jax import lax
from jax.experimental import pallas as pl
from jax.experimental.pallas import tpu as pltpu
from jax.experimental.pallas import tpu_sc as plsc

BATCH = 16384
EMB_D = 32

NUM_CORES = 2
NUM_SUBCORES = 16
NW = NUM_CORES * NUM_SUBCORES


B_PER_W = BATCH // NW
NRING = 16
NGRP = B_PER_W // NRING
LANE = 16


def _gather_one_table(tbl, out_hbm, idx, base, ring, obuf, sems):
    iota = lax.iota(jnp.int32, LANE)
    dlow = [iota] * LANE
    v0 = idx[pl.ds(0, LANE)]
    for b in range(LANE):
        ra = (v0[b] // 128) * 128
        pltpu.async_copy(tbl.at[:, pl.ds(ra, 128)], ring.at[b], sems.at[b])

    def body(g, vcur):
        vnext = idx[pl.ds((g + 1) * LANE, LANE)]
        k0 = g * LANE
        for b in range(LANE):
            pltpu.make_async_copy(
                tbl.at[:, pl.ds(0, 128)], ring.at[b], sems.at[b]).wait()
            c = vcur[b] % 128
            bv = jnp.full((LANE,), b, jnp.int32)
            cv = jnp.full((LANE,), c, jnp.int32)
            lo = plsc.load_gather(ring, [bv, iota, cv])
            hi = plsc.load_gather(ring, [bv, iota + 16, cv])
            kv = jnp.full((LANE,), k0 + b, jnp.int32)
            plsc.store_scatter(obuf, [iota, kv], lo)
            plsc.store_scatter(obuf, [iota + 16, kv], hi)

            @pl.when(g < NGRP - 1)
            def _():
                ra = (vnext[b] // 128) * 128
                pltpu.async_copy(tbl.at[:, pl.ds(ra, 128)], ring.at[b],
                                 sems.at[b])
        return vnext

    lax.fori_loop(0, NGRP, body, v0)
    pltpu.sync_copy(obuf, out_hbm.at[:, pl.ds(base, B_PER_W)])


def _sc_gather_body(users_hbm, items_hbm, gu_t, gi_t, mu_t, mi_t,
                    gu_o, gi_o, mu_o, mi_o,
                    iu, ii, ring, obuf, sems):
    w = lax.axis_index("s") * NUM_CORES + lax.axis_index("c")
    base = w * B_PER_W
    pltpu.sync_copy(users_hbm.at[pl.ds(base, B_PER_W)], iu.at[pl.ds(0, B_PER_W)])
    pltpu.sync_copy(items_hbm.at[pl.ds(base, B_PER_W)], ii.at[pl.ds(0, B_PER_W)])
    _gather_one_table(gu_t, gu_o, iu, base, ring, obuf, sems)
    _gather_one_table(gi_t, gi_o, ii, base, ring, obuf, sems)
    _gather_one_table(mu_t, mu_o, iu, base, ring, obuf, sems)
    _gather_one_table(mi_t, mi_o, ii, base, ring, obuf, sems)


@jax.jit
def _sc_gather(users, items, gu_t, gi_t, mu_t, mi_t):
    outT = jax.ShapeDtypeStruct((EMB_D, BATCH), jnp.float32)
    mesh = plsc.VectorSubcoreMesh(core_axis_name="c", subcore_axis_name="s")
    f = pl.kernel(
        _sc_gather_body,
        out_type=(outT, outT, outT, outT),
        mesh=mesh,
        compiler_params=pltpu.CompilerParams(needs_layout_passes=False),
        scratch_types=[
            pltpu.VMEM((B_PER_W + LANE,), jnp.int32),
            pltpu.VMEM((B_PER_W + LANE,), jnp.int32),
            pltpu.VMEM((NRING, EMB_D, 128), jnp.float32),
            pltpu.VMEM((EMB_D, B_PER_W), jnp.float32),
            pltpu.SemaphoreType.DMA((NRING,)),
        ],
    )
    return f(users, items, gu_t, gi_t, mu_t, mi_t)


def _tc_dense_body(guT, giT, muT, miT, W1, b1, W2, b2, Wf, bf, out):
    dn = (((0,), (0,)), ((), ()))
    cat = jnp.concatenate([muT[...], miT[...]], axis=0)
    x1 = lax.dot_general(W1[...], cat, dn,
                         preferred_element_type=jnp.float32)
    x1 = x1 + b1[...][:, None]
    h = jnp.maximum(x1, 0.0)
    m2 = lax.dot_general(W2[...], h, dn,
                         preferred_element_type=jnp.float32)
    m2 = m2 + b2[...][:, None]
    prod = guT[...] * giT[...]
    logit = (lax.dot_general(Wf[0:32, :], prod, dn,
                             preferred_element_type=jnp.float32)
             + lax.dot_general(Wf[32:96, :], m2, dn,
                               preferred_element_type=jnp.float32)
             + bf[...])
    out[...] = jax.nn.sigmoid(logit)


@jax.jit
def _tc_dense(guT, giT, muT, miT, W1, b1, W2, b2, Wf, bf):
    BLK = 2048
    grid = (BATCH // BLK,)
    t_spec = pl.BlockSpec((EMB_D, BLK), lambda i: (0, i))
    full = lambda shape: pl.BlockSpec(shape, lambda i: tuple(0 for _ in shape))
    return pl.pallas_call(
        _tc_dense_body,
        grid=grid,
        in_specs=[
            t_spec, t_spec, t_spec, t_spec,
            full((64, 128)), full((128,)), full((128, 64)), full((64,)),
            full((96, 1)), full((1,)),
        ],
        out_specs=pl.BlockSpec((1, BLK), lambda i: (0, i)),
        out_shape=jax.ShapeDtypeStruct((1, BATCH), jnp.float32),
    )(guT, giT, muT, miT, W1, b1, W2, b2, Wf, bf)


def kernel(X, gmf_user_emb, gmf_item_emb, mlp_user_emb, mlp_item_emb,
           W1, b1, W2, b2, Wf, bf):
    users = X[:, 0].astype(jnp.int32)
    items = X[:, 1].astype(jnp.int32)
    guT, giT, muT, miT = _sc_gather(users, items,
                                    gmf_user_emb.T, gmf_item_emb.T,
                                    mlp_user_emb.T, mlp_item_emb.T)
    out = _tc_dense(guT, giT, muT, miT, W1, b1, W2, b2, Wf, bf)
    return out.reshape(BATCH, 1)


# speedup vs baseline: 3.8792x; 1.0005x over previous
import jax
import jax.numpy as jnp
from jax import lax
from jax.experimental import pallas as pl
from jax.experimental.pallas import tpu as pltpu
from jax.experimental.pallas import tpu_sc as plsc

BATCH = 16384
EMB_D = 32

NUM_CORES = 2
NUM_SUBCORES = 16
NW = NUM_CORES * NUM_SUBCORES


B_PER_W = BATCH // NW
NRING = 16
NGRP = B_PER_W // NRING
LANE = 16


def _gather_one_table(tbl, out_hbm, idx, base, ring, obuf, sems):
    iota = lax.iota(jnp.int32, LANE)

    def fire(b, ra):
        for i in range(4):
            pltpu.async_copy(tbl.at[i, :, pl.ds(ra, 128)],
                             ring.at[b].at[pl.ds(i * 8, 8)], sems.at[b])

    v0 = idx[pl.ds(0, LANE)]
    for b in range(LANE):
        fire(b, (v0[b] // 128) * 128)

    def body(g, vcur):
        vnext = idx[pl.ds((g + 1) * LANE, LANE)]
        k0 = g * LANE
        for b in range(LANE):
            for i in range(4):
                pltpu.make_async_copy(
                    tbl.at[0, :, pl.ds(0, 128)],
                    ring.at[b].at[pl.ds(0, 8)], sems.at[b]).wait()
            c = vcur[b] % 128
            bv = jnp.full((LANE,), b, jnp.int32)
            cv = jnp.full((LANE,), c, jnp.int32)
            lo = plsc.load_gather(ring, [bv, iota, cv])
            hi = plsc.load_gather(ring, [bv, iota + 16, cv])
            kv = jnp.full((LANE,), k0 + b, jnp.int32)
            plsc.store_scatter(obuf, [iota, kv], lo)
            plsc.store_scatter(obuf, [iota + 16, kv], hi)

            @pl.when(g < NGRP - 1)
            def _():
                fire(b, (vnext[b] // 128) * 128)
        return vnext

    lax.fori_loop(0, NGRP, body, v0)
    pltpu.sync_copy(obuf, out_hbm.at[:, pl.ds(base, B_PER_W)])


def _sc_gather_body(users_hbm, items_hbm, gu_t, gi_t, mu_t, mi_t,
                    gu_o, gi_o, mu_o, mi_o,
                    iu, ii, ring, obuf, sems):
    w = lax.axis_index("s") * NUM_CORES + lax.axis_index("c")
    base = w * B_PER_W
    pltpu.sync_copy(users_hbm.at[pl.ds(base, B_PER_W)], iu.at[pl.ds(0, B_PER_W)])
    pltpu.sync_copy(items_hbm.at[pl.ds(base, B_PER_W)], ii.at[pl.ds(0, B_PER_W)])
    _gather_one_table(gu_t, gu_o, iu, base, ring, obuf, sems)
    _gather_one_table(gi_t, gi_o, ii, base, ring, obuf, sems)
    _gather_one_table(mu_t, mu_o, iu, base, ring, obuf, sems)
    _gather_one_table(mi_t, mi_o, ii, base, ring, obuf, sems)


@jax.jit
def _sc_gather(users, items, gu_t, gi_t, mu_t, mi_t):
    outT = jax.ShapeDtypeStruct((EMB_D, BATCH), jnp.float32)
    mesh = plsc.VectorSubcoreMesh(core_axis_name="c", subcore_axis_name="s")
    f = pl.kernel(
        _sc_gather_body,
        out_type=(outT, outT, outT, outT),
        mesh=mesh,
        compiler_params=pltpu.CompilerParams(needs_layout_passes=False),
        scratch_types=[
            pltpu.VMEM((B_PER_W + LANE,), jnp.int32),
            pltpu.VMEM((B_PER_W + LANE,), jnp.int32),
            pltpu.VMEM((NRING, EMB_D, 128), jnp.float32),
            pltpu.VMEM((EMB_D, B_PER_W), jnp.float32),
            pltpu.SemaphoreType.DMA((NRING,)),
        ],
    )
    return f(users, items, gu_t, gi_t, mu_t, mi_t)


def _tc_dense_body(guT, giT, muT, miT, W1, b1, W2, b2, Wf, bf, out):
    dn = (((0,), (0,)), ((), ()))
    cat = jnp.concatenate([muT[...], miT[...]], axis=0)
    x1 = lax.dot_general(W1[...], cat, dn,
                         preferred_element_type=jnp.float32)
    x1 = x1 + b1[...][:, None]
    h = jnp.maximum(x1, 0.0)
    m2 = lax.dot_general(W2[...], h, dn,
                         preferred_element_type=jnp.float32)
    m2 = m2 + b2[...][:, None]
    prod = guT[...] * giT[...]
    logit = (lax.dot_general(Wf[0:32, :], prod, dn,
                             preferred_element_type=jnp.float32)
             + lax.dot_general(Wf[32:96, :], m2, dn,
                               preferred_element_type=jnp.float32)
             + bf[...])
    out[...] = jax.nn.sigmoid(logit)


@jax.jit
def _tc_dense(guT, giT, muT, miT, W1, b1, W2, b2, Wf, bf):
    BLK = 2048
    grid = (BATCH // BLK,)
    t_spec = pl.BlockSpec((EMB_D, BLK), lambda i: (0, i))
    full = lambda shape: pl.BlockSpec(shape, lambda i: tuple(0 for _ in shape))
    return pl.pallas_call(
        _tc_dense_body,
        grid=grid,
        in_specs=[
            t_spec, t_spec, t_spec, t_spec,
            full((64, 128)), full((128,)), full((128, 64)), full((64,)),
            full((96, 1)), full((1,)),
        ],
        out_specs=pl.BlockSpec((1, BLK), lambda i: (0, i)),
        out_shape=jax.ShapeDtypeStruct((1, BATCH), jnp.float32),
    )(guT, giT, muT, miT, W1, b1, W2, b2, Wf, bf)


def kernel(X, gmf_user_emb, gmf_item_emb, mlp_user_emb, mlp_item_emb,
           W1, b1, W2, b2, Wf, bf):
    users = X[:, 0].astype(jnp.int32)
    items = X[:, 1].astype(jnp.int32)
    N = gmf_user_emb.shape[0]
    guT, giT, muT, miT = _sc_gather(
        users, items,
        gmf_user_emb.T.reshape(4, 8, N), gmf_item_emb.T.reshape(4, 8, N),
        mlp_user_emb.T.reshape(4, 8, N), mlp_item_emb.T.reshape(4, 8, N))
    out = _tc_dense(guT, giT, muT, miT, W1, b1, W2, b2, Wf, bf)
    return out.reshape(BATCH, 1)


# speedup vs baseline: 3.8940x; 1.0038x over previous
import jax
import jax.numpy as jnp
from jax import lax
from jax.experimental import pallas as pl
from jax.experimental.pallas import tpu as pltpu
from jax.experimental.pallas import tpu_sc as plsc

BATCH = 16384
EMB_D = 32

NUM_CORES = 2
NUM_SUBCORES = 16
NW = NUM_CORES * NUM_SUBCORES


B_PER_W = BATCH // NW
NRING = 16
NGRP = B_PER_W // NRING
LANE = 16


def _gather_one_table(tbl, out_hbm, idx, base, ring, obuf, sems):
    iota = lax.iota(jnp.int32, LANE)

    def fire(b, ra):
        for i in range(4):
            pltpu.async_copy(tbl.at[i, :, pl.ds(ra, 128)],
                             ring.at[b].at[pl.ds(i * 8, 8)], sems.at[b])

    v0 = idx[pl.ds(0, LANE)]
    for b in range(LANE):
        fire(b, (v0[b] // 128) * 128)

    def body(g, vcur):
        vnext = idx[pl.ds((g + 1) * LANE, LANE)]
        k0 = g * LANE
        for b in range(LANE):
            for i in range(4):
                pltpu.make_async_copy(
                    tbl.at[0, :, pl.ds(0, 128)],
                    ring.at[b].at[pl.ds(0, 8)], sems.at[b]).wait()
            c = vcur[b] % 128
            bv = jnp.full((LANE,), b, jnp.int32)
            cv = jnp.full((LANE,), c, jnp.int32)
            lo = plsc.load_gather(ring, [bv, iota, cv])
            hi = plsc.load_gather(ring, [bv, iota + 16, cv])
            kv = jnp.full((LANE,), k0 + b, jnp.int32)
            plsc.store_scatter(obuf, [iota, kv], lo)
            plsc.store_scatter(obuf, [iota + 16, kv], hi)

            @pl.when(g < NGRP - 1)
            def _():
                fire(b, (vnext[b] // 128) * 128)
        return vnext

    lax.fori_loop(0, NGRP, body, v0)
    pltpu.sync_copy(obuf, out_hbm.at[:, pl.ds(base, B_PER_W)])


def _sc_gather_body(users_hbm, items_hbm, gu_t, gi_t, mu_t, mi_t,
                    gu_o, gi_o, mu_o, mi_o,
                    iu, ii, ring, obuf, sems):
    w = lax.axis_index("s") * NUM_CORES + lax.axis_index("c")
    base = w * B_PER_W
    pltpu.sync_copy(users_hbm.at[pl.ds(base, B_PER_W)], iu.at[pl.ds(0, B_PER_W)])
    pltpu.sync_copy(items_hbm.at[pl.ds(base, B_PER_W)], ii.at[pl.ds(0, B_PER_W)])
    _gather_one_table(gu_t, gu_o, iu, base, ring, obuf, sems)
    _gather_one_table(gi_t, gi_o, ii, base, ring, obuf, sems)
    _gather_one_table(mu_t, mu_o, iu, base, ring, obuf, sems)
    _gather_one_table(mi_t, mi_o, ii, base, ring, obuf, sems)


@jax.jit
def _sc_gather(users, items, gu_t, gi_t, mu_t, mi_t):
    outT = jax.ShapeDtypeStruct((EMB_D, BATCH), jnp.float32)
    mesh = plsc.VectorSubcoreMesh(core_axis_name="c", subcore_axis_name="s")
    f = pl.kernel(
        _sc_gather_body,
        out_type=(outT, outT, outT, outT),
        mesh=mesh,
        compiler_params=pltpu.CompilerParams(needs_layout_passes=False),
        scratch_types=[
            pltpu.VMEM((B_PER_W + LANE,), jnp.int32),
            pltpu.VMEM((B_PER_W + LANE,), jnp.int32),
            pltpu.VMEM((NRING, EMB_D, 128), jnp.float32),
            pltpu.VMEM((EMB_D, B_PER_W), jnp.float32),
            pltpu.SemaphoreType.DMA((NRING,)),
        ],
    )
    return f(users, items, gu_t, gi_t, mu_t, mi_t)


def _tc_dense_body(guT, giT, muT, miT, W1, b1, W2, b2, Wf, bf, out):
    dn = (((0,), (0,)), ((), ()))
    cat = jnp.concatenate([muT[...], miT[...]], axis=0)
    x1 = lax.dot_general(W1[...], cat, dn,
                         preferred_element_type=jnp.float32)
    x1 = x1 + b1[...][:, None]
    h = jnp.maximum(x1, 0.0)
    m2 = lax.dot_general(W2[...], h, dn,
                         preferred_element_type=jnp.float32)
    m2 = m2 + b2[...][:, None]
    prod = guT[...] * giT[...]
    logit = (lax.dot_general(Wf[0:32, :], prod, dn,
                             preferred_element_type=jnp.float32)
             + lax.dot_general(Wf[32:96, :], m2, dn,
                               preferred_element_type=jnp.float32)
             + bf[...])
    out[...] = jax.nn.sigmoid(logit)


@jax.jit
def _tc_dense(guT, giT, muT, miT, W1, b1, W2, b2, Wf, bf):
    BLK = 4096
    grid = (BATCH // BLK,)
    t_spec = pl.BlockSpec((EMB_D, BLK), lambda i: (0, i))
    full = lambda shape: pl.BlockSpec(shape, lambda i: tuple(0 for _ in shape))
    return pl.pallas_call(
        _tc_dense_body,
        grid=grid,
        in_specs=[
            t_spec, t_spec, t_spec, t_spec,
            full((64, 128)), full((128,)), full((128, 64)), full((64,)),
            full((96, 1)), full((1,)),
        ],
        out_specs=pl.BlockSpec((1, BLK), lambda i: (0, i)),
        out_shape=jax.ShapeDtypeStruct((1, BATCH), jnp.float32),
    )(guT, giT, muT, miT, W1, b1, W2, b2, Wf, bf)


def kernel(X, gmf_user_emb, gmf_item_emb, mlp_user_emb, mlp_item_emb,
           W1, b1, W2, b2, Wf, bf):
    users = X[:, 0].astype(jnp.int32)
    items = X[:, 1].astype(jnp.int32)
    N = gmf_user_emb.shape[0]
    guT, giT, muT, miT = _sc_gather(
        users, items,
        gmf_user_emb.T.reshape(4, 8, N), gmf_item_emb.T.reshape(4, 8, N),
        mlp_user_emb.T.reshape(4, 8, N), mlp_item_emb.T.reshape(4, 8, N))
    out = _tc_dense(guT, giT, muT, miT, W1, b1, W2, b2, Wf, bf)
    return out.reshape(BATCH, 1)


# speedup vs baseline: 3.9095x; 1.0040x over previous
import jax
import jax.numpy as jnp
from jax import lax
from jax.experimental import pallas as pl
from jax.experimental.pallas import tpu as pltpu
from jax.experimental.pallas import tpu_sc as plsc

BATCH = 16384
EMB_D = 32

NUM_CORES = 2
NUM_SUBCORES = 16
NW = NUM_CORES * NUM_SUBCORES


B_PER_W = BATCH // NW
NRING = 16
NGRP = B_PER_W // NRING
LANE = 16


def _gather_one_table(tbl, out_hbm, idx, base, ring, obuf, sems, osem,
                      drain_prev):
    iota = lax.iota(jnp.int32, LANE)
    if drain_prev:
        pltpu.make_async_copy(
            obuf, out_hbm.at[:, pl.ds(base, B_PER_W)], osem).wait()

    def fire(b, ra):
        for i in range(4):
            pltpu.async_copy(tbl.at[i, :, pl.ds(ra, 128)],
                             ring.at[b].at[pl.ds(i * 8, 8)], sems.at[b])

    v0 = idx[pl.ds(0, LANE)]
    for b in range(LANE):
        fire(b, (v0[b] // 128) * 128)

    def body(g, vcur):
        vnext = idx[pl.ds((g + 1) * LANE, LANE)]
        k0 = g * LANE
        for b in range(LANE):
            for i in range(4):
                pltpu.make_async_copy(
                    tbl.at[0, :, pl.ds(0, 128)],
                    ring.at[b].at[pl.ds(0, 8)], sems.at[b]).wait()
            c = vcur[b] % 128
            bv = jnp.full((LANE,), b, jnp.int32)
            cv = jnp.full((LANE,), c, jnp.int32)
            lo = plsc.load_gather(ring, [bv, iota, cv])
            hi = plsc.load_gather(ring, [bv, iota + 16, cv])
            kv = jnp.full((LANE,), k0 + b, jnp.int32)
            plsc.store_scatter(obuf, [iota, kv], lo)
            plsc.store_scatter(obuf, [iota + 16, kv], hi)

            @pl.when(g < NGRP - 1)
            def _():
                fire(b, (vnext[b] // 128) * 128)
        return vnext

    lax.fori_loop(0, NGRP, body, v0)
    pltpu.async_copy(obuf, out_hbm.at[:, pl.ds(base, B_PER_W)], osem)


def _sc_gather_body(users_hbm, items_hbm, gu_t, gi_t, mu_t, mi_t,
                    gu_o, gi_o, mu_o, mi_o,
                    iu, ii, ring, ob0, ob1, sems, os0, os1):
    w = lax.axis_index("s") * NUM_CORES + lax.axis_index("c")
    base = w * B_PER_W
    pltpu.sync_copy(users_hbm.at[pl.ds(base, B_PER_W)], iu.at[pl.ds(0, B_PER_W)])
    pltpu.sync_copy(items_hbm.at[pl.ds(base, B_PER_W)], ii.at[pl.ds(0, B_PER_W)])
    _gather_one_table(gu_t, gu_o, iu, base, ring, ob0, sems, os0, False)
    _gather_one_table(gi_t, gi_o, ii, base, ring, ob1, sems, os1, False)
    _gather_one_table(mu_t, mu_o, iu, base, ring, ob0, sems, os0, True)
    _gather_one_table(mi_t, mi_o, ii, base, ring, ob1, sems, os1, True)
    pltpu.make_async_copy(ob0, mu_o.at[:, pl.ds(base, B_PER_W)], os0).wait()
    pltpu.make_async_copy(ob1, mi_o.at[:, pl.ds(base, B_PER_W)], os1).wait()


@jax.jit
def _sc_gather(users, items, gu_t, gi_t, mu_t, mi_t):
    outT = jax.ShapeDtypeStruct((EMB_D, BATCH), jnp.float32)
    mesh = plsc.VectorSubcoreMesh(core_axis_name="c", subcore_axis_name="s")
    f = pl.kernel(
        _sc_gather_body,
        out_type=(outT, outT, outT, outT),
        mesh=mesh,
        compiler_params=pltpu.CompilerParams(needs_layout_passes=False),
        scratch_types=[
            pltpu.VMEM((B_PER_W + LANE,), jnp.int32),
            pltpu.VMEM((B_PER_W + LANE,), jnp.int32),
            pltpu.VMEM((NRING, EMB_D, 128), jnp.float32),
            pltpu.VMEM((EMB_D, B_PER_W), jnp.float32),
            pltpu.VMEM((EMB_D, B_PER_W), jnp.float32),
            pltpu.SemaphoreType.DMA((NRING,)),
            pltpu.SemaphoreType.DMA,
            pltpu.SemaphoreType.DMA,
        ],
    )
    return f(users, items, gu_t, gi_t, mu_t, mi_t)


def _tc_dense_body(guT, giT, muT, miT, W1, b1, W2, b2, Wf, bf, out):
    dn = (((0,), (0,)), ((), ()))
    cat = jnp.concatenate([muT[...], miT[...]], axis=0)
    x1 = lax.dot_general(W1[...], cat, dn,
                         preferred_element_type=jnp.float32)
    x1 = x1 + b1[...][:, None]
    h = jnp.maximum(x1, 0.0)
    m2 = lax.dot_general(W2[...], h, dn,
                         preferred_element_type=jnp.float32)
    m2 = m2 + b2[...][:, None]
    prod = guT[...] * giT[...]
    logit = (lax.dot_general(Wf[0:32, :], prod, dn,
                             preferred_element_type=jnp.float32)
             + lax.dot_general(Wf[32:96, :], m2, dn,
                               preferred_element_type=jnp.float32)
             + bf[...])
    out[...] = jax.nn.sigmoid(logit)


@jax.jit
def _tc_dense(guT, giT, muT, miT, W1, b1, W2, b2, Wf, bf):
    BLK = 4096
    grid = (BATCH // BLK,)
    t_spec = pl.BlockSpec((EMB_D, BLK), lambda i: (0, i))
    full = lambda shape: pl.BlockSpec(shape, lambda i: tuple(0 for _ in shape))
    return pl.pallas_call(
        _tc_dense_body,
        grid=grid,
        in_specs=[
            t_spec, t_spec, t_spec, t_spec,
            full((64, 128)), full((128,)), full((128, 64)), full((64,)),
            full((96, 1)), full((1,)),
        ],
        out_specs=pl.BlockSpec((1, BLK), lambda i: (0, i)),
        out_shape=jax.ShapeDtypeStruct((1, BATCH), jnp.float32),
    )(guT, giT, muT, miT, W1, b1, W2, b2, Wf, bf)


def kernel(X, gmf_user_emb, gmf_item_emb, mlp_user_emb, mlp_item_emb,
           W1, b1, W2, b2, Wf, bf):
    users = X[:, 0].astype(jnp.int32)
    items = X[:, 1].astype(jnp.int32)
    N = gmf_user_emb.shape[0]
    guT, giT, muT, miT = _sc_gather(
        users, items,
        gmf_user_emb.T.reshape(4, 8, N), gmf_item_emb.T.reshape(4, 8, N),
        mlp_user_emb.T.reshape(4, 8, N), mlp_item_emb.T.reshape(4, 8, N))
    out = _tc_dense(guT, giT, muT, miT, W1, b1, W2, b2, Wf, bf)
    return out.reshape(BATCH, 1)
